# baseline (speedup 1.0000x reference)
import jax
import jax.numpy as jnp
from jax.experimental import pallas as pl
from jax.experimental.pallas import tpu as pltpu

_K = 5
_BQ = 256


def _simtopk_kernel(q_ref, k_ref, idx_ref):
    q = q_ref[...]
    k = k_ref[...]
    qn = q / jnp.maximum(jnp.sqrt(jnp.sum(q * q, axis=1, keepdims=True)), 1e-12)
    kn = k / jnp.maximum(jnp.sqrt(jnp.sum(k * k, axis=1, keepdims=True)), 1e-12)
    sim = jnp.dot(qn, kn.T, preferred_element_type=jnp.float32)
    cols = jax.lax.broadcasted_iota(jnp.int32, sim.shape, 1)
    picks = []
    for _ in range(_K):
        m = jnp.max(sim, axis=1, keepdims=True)
        a = jnp.min(jnp.where(sim == m, cols, jnp.int32(2**30)), axis=1)
        picks.append(a)
        sim = jnp.where(cols == a[:, None], -jnp.inf, sim)
    idx_ref[...] = jnp.stack(picks, axis=1)


def _gather_kernel(idx_ref, src_ref, out_ref):
    del idx_ref
    out_ref[...] = src_ref[...]


def kernel(query, top_k, prompts, prompt_keys):
    del top_k
    nq, d = query.shape
    n, k, _ = prompts.shape

    indices = pl.pallas_call(
        _simtopk_kernel,
        grid=(nq // _BQ,),
        in_specs=[
            pl.BlockSpec((_BQ, d), lambda i: (i, 0)),
            pl.BlockSpec((n, d), lambda i: (0, 0)),
        ],
        out_specs=pl.BlockSpec((_BQ, _K), lambda i: (i, 0)),
        out_shape=jax.ShapeDtypeStruct((nq, _K), jnp.int32),
    )(query, prompt_keys)

    flat_idx = indices.reshape(-1)
    gathered = pl.pallas_call(
        _gather_kernel,
        grid_spec=pltpu.PrefetchScalarGridSpec(
            num_scalar_prefetch=1,
            grid=(nq * _K,),
            in_specs=[
                pl.BlockSpec((1, k, d), lambda i, idx_ref: (idx_ref[i], 0, 0)),
            ],
            out_specs=pl.BlockSpec((1, k, d), lambda i, idx_ref: (i, 0, 0)),
        ),
        out_shape=jax.ShapeDtypeStruct((nq * _K, k, d), jnp.float32),
    )(flat_idx, prompts)

    return gathered.reshape(nq, _K, k, d), indices


# speedup vs baseline: 16.2023x; 16.2023x over previous
import jax
import jax.numpy as jnp
from jax.experimental import pallas as pl
from jax.experimental.pallas import tpu as pltpu

_K = 5
_BQ = 256


def _simtopk_kernel(q_ref, k_ref, idx_ref):
    q = q_ref[...]
    k = k_ref[...]
    qn = q / jnp.maximum(jnp.sqrt(jnp.sum(q * q, axis=1, keepdims=True)), 1e-12)
    kn = k / jnp.maximum(jnp.sqrt(jnp.sum(k * k, axis=1, keepdims=True)), 1e-12)
    sim = jnp.dot(qn, kn.T, preferred_element_type=jnp.float32)
    cols = jax.lax.broadcasted_iota(jnp.int32, sim.shape, 1)
    picks = []
    for _ in range(_K):
        m = jnp.max(sim, axis=1, keepdims=True)
        a = jnp.min(jnp.where(sim == m, cols, jnp.int32(2**30)), axis=1)
        picks.append(a)
        sim = jnp.where(cols == a[:, None], -jnp.inf, sim)
    idx_ref[...] = jnp.stack(picks, axis=1)


_BG = 32


def _gather_kernel(idx_ref, src_ref, out_ref):
    i = pl.program_id(0)
    for q in range(_BG):
        for j in range(_K):
            row = idx_ref[i * _BG + q, j]
            out_ref[q, j] = src_ref[row]


def kernel(query, top_k, prompts, prompt_keys):
    del top_k
    nq, d = query.shape
    n, k, _ = prompts.shape

    indices = pl.pallas_call(
        _simtopk_kernel,
        grid=(nq // _BQ,),
        in_specs=[
            pl.BlockSpec((_BQ, d), lambda i: (i, 0)),
            pl.BlockSpec((n, d), lambda i: (0, 0)),
        ],
        out_specs=pl.BlockSpec((_BQ, _K), lambda i: (i, 0)),
        out_shape=jax.ShapeDtypeStruct((nq, _K), jnp.int32),
    )(query, prompt_keys)

    gathered = pl.pallas_call(
        _gather_kernel,
        grid_spec=pltpu.PrefetchScalarGridSpec(
            num_scalar_prefetch=1,
            grid=(nq // _BG,),
            in_specs=[
                pl.BlockSpec((n, k, d), lambda i, idx_ref: (0, 0, 0)),
            ],
            out_specs=pl.BlockSpec((_BG, _K, k, d), lambda i, idx_ref: (i, 0, 0, 0)),
        ),
        out_shape=jax.ShapeDtypeStruct((nq, _K, k, d), jnp.float32),
    )(indices, prompts)

    return gathered, indices
